# baseline (speedup 1.0000x reference)
import functools

import jax
import jax.numpy as jnp
import numpy as np
from jax import lax
from jax.experimental import pallas as pl
from jax.experimental.pallas import tpu as pltpu
from jax.experimental.pallas import tpu_sc as plsc

DIM = 64
MAX_NORM = 1.0


_Q = 2048

_HI = np.uint32(0xFFFF0000)


def _sel(parity):
    d = lax.broadcasted_iota(jnp.int32, (DIM, DIM // 2), 0)
    c = lax.broadcasted_iota(jnp.int32, (DIM, DIM // 2), 1)
    return (d == 2 * c + parity).astype(jnp.float32)


def _pack_bf16_pairs(even_f32, odd_f32):
    eb = even_f32.astype(jnp.bfloat16).astype(jnp.float32)
    ob = odd_f32.astype(jnp.bfloat16).astype(jnp.float32)
    ue = lax.bitcast_convert_type(eb, jnp.uint32)
    uo = lax.bitcast_convert_type(ob, jnp.uint32)
    packed = jnp.bitwise_or(jnp.right_shift(ue, 16),
                            jnp.bitwise_and(uo, _HI))
    return lax.bitcast_convert_type(packed, jnp.float32)


def _renorm_body(tt_ref, out_ref):
    x = tt_ref[...]
    norm = jnp.sqrt(jnp.sum(x * x, axis=0, keepdims=True))
    scale = jnp.where(norm > MAX_NORM, MAX_NORM / (norm + 1e-7), 1.0)
    y = x * scale
    se, so = _sel(0), _sel(1)
    dn = (((0,), (0,)), ((), ()))
    quarters = []
    for j in range(4):
        q = y[:, j * _Q:(j + 1) * _Q]
        ye = lax.dot_general(q, se, dn, preferred_element_type=jnp.float32)
        yo = lax.dot_general(q, so, dn, preferred_element_type=jnp.float32)
        quarters.append(_pack_bf16_pairs(ye, yo))
    out_ref[...] = jnp.concatenate(quarters, axis=1)


def _renorm_flat(table):
    vocab = table.shape[0]
    g = (vocab + 4 * _Q - 1) // (4 * _Q)
    sf = pl.pallas_call(
        _renorm_body,
        grid=(g,),
        in_specs=[pl.BlockSpec((DIM, 4 * _Q), lambda i: (0, i))],
        out_specs=pl.BlockSpec((_Q, 2 * DIM), lambda i: (i, 0)),
        out_shape=jax.ShapeDtypeStruct((g * _Q, 2 * DIM), jnp.float32),
    )(table.T)
    return sf.reshape(g * _Q * 4, DIM // 2)


_CHUNK = 128
_NBUF = 6
_LAG = _NBUF // 2


@functools.cache
def _make_gather(bsz, seq, row_off):
    info = plsc.get_sparse_core_info()
    nc, ns = info.num_cores, info.num_subcores
    nw = nc * ns
    bblks = bsz // _CHUNK
    qrows = bsz // 4
    per_w = bblks * seq // nw
    assert per_w * nw == bblks * seq and bblks * _CHUNK == bsz
    assert bblks & (bblks - 1) == 0
    sh_l = bblks.bit_length() - 1
    sh_b = _CHUNK.bit_length() - 1
    sh_q = qrows.bit_length() - 1
    mesh = plsc.VectorSubcoreMesh(core_axis_name="c", subcore_axis_name="s")

    @functools.partial(
        pl.kernel,
        mesh=mesh,
        compiler_params=pltpu.CompilerParams(
            use_tc_tiling_on_sc=False, needs_layout_passes=False),
        out_type=jax.ShapeDtypeStruct((seq * qrows, 2 * DIM), jnp.float32),
        scratch_types=(
            [pltpu.VMEM((per_w, _CHUNK), jnp.int32)]
            + [pltpu.VMEM((_CHUNK, DIM // 2), jnp.float32)
               for _ in range(_NBUF)]
            + [pltpu.SemaphoreType.DMA for _ in range(2 * _NBUF)]
        ),
    )
    def gather_k(tab_hbm, idx_hbm, out_hbm, idx_v, *rest):
        bufs = rest[:_NBUF]
        gsems = rest[_NBUF:2 * _NBUF]
        wsems = rest[2 * _NBUF:]
        wid = lax.axis_index("s") * nc + lax.axis_index("c")
        ubase = wid * per_w
        pltpu.sync_copy(
            idx_hbm.at[pl.ds(row_off + wid * per_w, per_w)], idx_v)

        qb = _Q.bit_length() - 1
        @pl.loop(0, per_w)
        def _(j):
            for k in range(_CHUNK // 16):
                v = idx_v[j, pl.ds(k * 16, 16)]
                g = jnp.right_shift(v, qb + 2)
                jj = jnp.bitwise_and(jnp.right_shift(v, qb), 3)
                p = jnp.bitwise_and(v, _Q - 1)
                idx_v[j, pl.ds(k * 16, 16)] = (
                    jnp.left_shift(g, qb + 2) + jnp.left_shift(p, 2) + jj)

        hg = [None] * _NBUF
        hw = [None] * _NBUF
        for j in range(per_w + _LAG):
            if j < per_w:
                b = j % _NBUF
                if j >= _NBUF:
                    hw[b].wait()
                hg[b] = pltpu.async_copy(
                    tab_hbm.at[idx_v.at[j]], bufs[b], gsems[b])
            i = j - _LAG
            if 0 <= i < per_w:
                bi = i % _NBUF
                hg[bi].wait()
                u = ubase + i
                li = jnp.right_shift(u, sh_l)
                b0 = jnp.left_shift(jnp.bitwise_and(u, bblks - 1), sh_b)
                seg = jnp.right_shift(b0, sh_q)
                row0 = pl.multiple_of(
                    li * qrows + jnp.bitwise_and(b0, qrows - 1), _CHUNK)
                col0 = pl.multiple_of(seg * (DIM // 2), DIM // 2)
                hw[bi] = pltpu.async_copy(
                    bufs[bi],
                    out_hbm.at[pl.ds(row0, _CHUNK), pl.ds(col0, DIM // 2)],
                    wsems[bi])
        for i in range(max(0, per_w - _NBUF), per_w):
            hw[i % _NBUF].wait()

    return gather_k


_PL3 = 5


def _interleave_mat():
    d = lax.broadcasted_iota(jnp.int32, (DIM, DIM), 0)
    c = lax.broadcasted_iota(jnp.int32, (DIM, DIM), 1)
    return (((c < 32) & (d == 2 * c))
            | ((c >= 32) & (d == 2 * (c - 32) + 1))).astype(jnp.float32)


def _plane_body(f_ref, *rest):
    out_ref = rest[-1]
    x = f_ref[...]
    qrows = x.shape[0] // _PL3
    u_mat = _interleave_mat()
    dn = (((1,), (1,)), ((), ()))
    planes = []
    for p in range(_PL3):
        xp = x[p * qrows:(p + 1) * qrows]
        segs = []
        for q in range(4):
            w = lax.bitcast_convert_type(
                xp[:, q * 32:(q + 1) * 32], jnp.uint32)
            lo = lax.bitcast_convert_type(
                jnp.left_shift(w, 16), jnp.float32)
            hi = lax.bitcast_convert_type(
                jnp.bitwise_and(w, _HI), jnp.float32)
            cat = jnp.concatenate([lo, hi], axis=1)
            segs.append(lax.dot_general(
                u_mat, cat, dn, preferred_element_type=jnp.float32))
        planes.append(jnp.concatenate(segs, axis=1)[None])
    out_ref[...] = jnp.concatenate(planes, axis=0)


def _plane_transpose(f, prev, l_off, seq, bsz):
    qrows = bsz // 4
    blk_off = l_off // _PL3
    in_specs = [pl.BlockSpec((_PL3 * qrows, 2 * DIM), lambda i: (i, 0))]
    args = (f,)
    aliases = {}
    if prev is not None:
        in_specs.append(pl.BlockSpec(memory_space=pl.ANY))
        args = (f, prev)
        aliases = {1: 0}
    return pl.pallas_call(
        _plane_body,
        grid=(f.shape[0] // (_PL3 * qrows),),
        in_specs=in_specs,
        out_specs=pl.BlockSpec(
            (_PL3, DIM, bsz), lambda i: (i + blk_off, 0, 0)),
        out_shape=jax.ShapeDtypeStruct((seq, DIM, bsz), jnp.float32),
        input_output_aliases=aliases,
    )(*args)


_SPLITS = 5


def kernel(doc, table):
    b, l = doc.shape
    flat = _renorm_flat(table)
    idx2d = doc.T.reshape(b * l // _CHUNK, _CHUNK)
    ls = l // _SPLITS
    rows_per_split = ls * b // _CHUNK
    fs = [_make_gather(b, ls, s * rows_per_split)(flat, idx2d)
          for s in range(_SPLITS)]
    out = None
    for s in range(_SPLITS):
        out = _plane_transpose(fs[s], out, s * ls, l, b)
    return jnp.transpose(out, (2, 0, 1))


# speedup vs baseline: 1.0441x; 1.0441x over previous
import functools

import jax
import jax.numpy as jnp
import numpy as np
from jax import lax
from jax.experimental import pallas as pl
from jax.experimental.pallas import tpu as pltpu
from jax.experimental.pallas import tpu_sc as plsc

DIM = 64
MAX_NORM = 1.0


_Q = 4096

_HI = np.uint32(0xFFFF0000)


def _sel(parity):
    d = lax.broadcasted_iota(jnp.int32, (DIM, DIM // 2), 0)
    c = lax.broadcasted_iota(jnp.int32, (DIM, DIM // 2), 1)
    return (d == 2 * c + parity).astype(jnp.float32)


def _pack_bf16_pairs(even_f32, odd_f32):
    eb = even_f32.astype(jnp.bfloat16).astype(jnp.float32)
    ob = odd_f32.astype(jnp.bfloat16).astype(jnp.float32)
    ue = lax.bitcast_convert_type(eb, jnp.uint32)
    uo = lax.bitcast_convert_type(ob, jnp.uint32)
    packed = jnp.bitwise_or(jnp.right_shift(ue, 16),
                            jnp.bitwise_and(uo, _HI))
    return lax.bitcast_convert_type(packed, jnp.float32)


def _renorm_body(tt_ref, out_ref):
    x = tt_ref[...]
    norm = jnp.sqrt(jnp.sum(x * x, axis=0, keepdims=True))
    scale = jnp.where(norm > MAX_NORM, MAX_NORM / (norm + 1e-7), 1.0)
    y = x * scale
    se, so = _sel(0), _sel(1)
    dn = (((0,), (0,)), ((), ()))
    quarters = []
    for j in range(4):
        q = y[:, j * _Q:(j + 1) * _Q]
        ye = lax.dot_general(q, se, dn, preferred_element_type=jnp.float32)
        yo = lax.dot_general(q, so, dn, preferred_element_type=jnp.float32)
        quarters.append(_pack_bf16_pairs(ye, yo))
    out_ref[...] = jnp.concatenate(quarters, axis=1)


def _renorm_flat(table):
    vocab = table.shape[0]
    g = (vocab + 4 * _Q - 1) // (4 * _Q)
    sf = pl.pallas_call(
        _renorm_body,
        grid=(g,),
        in_specs=[pl.BlockSpec((DIM, 4 * _Q), lambda i: (0, i))],
        out_specs=pl.BlockSpec((_Q, 2 * DIM), lambda i: (i, 0)),
        out_shape=jax.ShapeDtypeStruct((g * _Q, 2 * DIM), jnp.float32),
    )(table.T)
    return sf.reshape(g * _Q * 4, DIM // 2)


_CHUNK = 128
_NBUF = 6
_LAG = _NBUF // 2


@functools.cache
def _make_gather(bsz, seq, row_off):
    info = plsc.get_sparse_core_info()
    nc, ns = info.num_cores, info.num_subcores
    nw = nc * ns
    bblks = bsz // _CHUNK
    qrows = bsz // 4
    per_w = bblks * seq // nw
    assert per_w * nw == bblks * seq and bblks * _CHUNK == bsz
    assert bblks & (bblks - 1) == 0
    sh_l = bblks.bit_length() - 1
    sh_b = _CHUNK.bit_length() - 1
    sh_q = qrows.bit_length() - 1
    mesh = plsc.VectorSubcoreMesh(core_axis_name="c", subcore_axis_name="s")

    @functools.partial(
        pl.kernel,
        mesh=mesh,
        compiler_params=pltpu.CompilerParams(
            use_tc_tiling_on_sc=False, needs_layout_passes=False),
        out_type=jax.ShapeDtypeStruct((seq * qrows, 2 * DIM), jnp.float32),
        scratch_types=(
            [pltpu.VMEM((per_w, _CHUNK), jnp.int32)]
            + [pltpu.VMEM((_CHUNK, DIM // 2), jnp.float32)
               for _ in range(_NBUF)]
            + [pltpu.SemaphoreType.DMA for _ in range(2 * _NBUF)]
        ),
    )
    def gather_k(tab_hbm, idx_hbm, out_hbm, idx_v, *rest):
        bufs = rest[:_NBUF]
        gsems = rest[_NBUF:2 * _NBUF]
        wsems = rest[2 * _NBUF:]
        wid = lax.axis_index("s") * nc + lax.axis_index("c")
        ubase = wid * per_w
        pltpu.sync_copy(
            idx_hbm.at[pl.ds(row_off + wid * per_w, per_w)], idx_v)

        qb = _Q.bit_length() - 1
        @pl.loop(0, per_w)
        def _(j):
            for k in range(_CHUNK // 16):
                v = idx_v[j, pl.ds(k * 16, 16)]
                g = jnp.right_shift(v, qb + 2)
                jj = jnp.bitwise_and(jnp.right_shift(v, qb), 3)
                p = jnp.bitwise_and(v, _Q - 1)
                idx_v[j, pl.ds(k * 16, 16)] = (
                    jnp.left_shift(g, qb + 2) + jnp.left_shift(p, 2) + jj)

        hg = [None] * _NBUF
        hw = [None] * _NBUF
        for j in range(per_w + _LAG):
            if j < per_w:
                b = j % _NBUF
                if j >= _NBUF:
                    hw[b].wait()
                hg[b] = pltpu.async_copy(
                    tab_hbm.at[idx_v.at[j]], bufs[b], gsems[b])
            i = j - _LAG
            if 0 <= i < per_w:
                bi = i % _NBUF
                hg[bi].wait()
                u = ubase + i
                li = jnp.right_shift(u, sh_l)
                b0 = jnp.left_shift(jnp.bitwise_and(u, bblks - 1), sh_b)
                seg = jnp.right_shift(b0, sh_q)
                row0 = pl.multiple_of(
                    li * qrows + jnp.bitwise_and(b0, qrows - 1), _CHUNK)
                col0 = pl.multiple_of(seg * (DIM // 2), DIM // 2)
                hw[bi] = pltpu.async_copy(
                    bufs[bi],
                    out_hbm.at[pl.ds(row0, _CHUNK), pl.ds(col0, DIM // 2)],
                    wsems[bi])
        for i in range(max(0, per_w - _NBUF), per_w):
            hw[i % _NBUF].wait()

    return gather_k


_PL3 = 5


def _interleave_mat():
    d = lax.broadcasted_iota(jnp.int32, (DIM, DIM), 0)
    c = lax.broadcasted_iota(jnp.int32, (DIM, DIM), 1)
    return (((c < 32) & (d == 2 * c))
            | ((c >= 32) & (d == 2 * (c - 32) + 1))).astype(jnp.float32)


def _plane_body(f_ref, *rest):
    out_ref = rest[-1]
    x = f_ref[...]
    qrows = x.shape[0] // _PL3
    u_mat = _interleave_mat()
    dn = (((1,), (1,)), ((), ()))
    planes = []
    for p in range(_PL3):
        xp = x[p * qrows:(p + 1) * qrows]
        segs = []
        for q in range(4):
            w = lax.bitcast_convert_type(
                xp[:, q * 32:(q + 1) * 32], jnp.uint32)
            lo = lax.bitcast_convert_type(
                jnp.left_shift(w, 16), jnp.float32)
            hi = lax.bitcast_convert_type(
                jnp.bitwise_and(w, _HI), jnp.float32)
            cat = jnp.concatenate([lo, hi], axis=1)
            segs.append(lax.dot_general(
                u_mat, cat, dn, preferred_element_type=jnp.float32))
        planes.append(jnp.concatenate(segs, axis=1)[None])
    out_ref[...] = jnp.concatenate(planes, axis=0)


def _plane_transpose(f, prev, l_off, seq, bsz):
    qrows = bsz // 4
    blk_off = l_off // _PL3
    in_specs = [pl.BlockSpec((_PL3 * qrows, 2 * DIM), lambda i: (i, 0))]
    args = (f,)
    aliases = {}
    if prev is not None:
        in_specs.append(pl.BlockSpec(memory_space=pl.ANY))
        args = (f, prev)
        aliases = {1: 0}
    return pl.pallas_call(
        _plane_body,
        grid=(f.shape[0] // (_PL3 * qrows),),
        in_specs=in_specs,
        out_specs=pl.BlockSpec(
            (_PL3, DIM, bsz), lambda i: (i + blk_off, 0, 0)),
        out_shape=jax.ShapeDtypeStruct((seq, DIM, bsz), jnp.float32),
        input_output_aliases=aliases,
    )(*args)


_SPLITS = 2


def kernel(doc, table):
    b, l = doc.shape
    flat = _renorm_flat(table)
    idx2d = doc.T.reshape(b * l // _CHUNK, _CHUNK)
    ls = l // _SPLITS
    rows_per_split = ls * b // _CHUNK
    fs = [_make_gather(b, ls, s * rows_per_split)(flat, idx2d)
          for s in range(_SPLITS)]
    out = None
    for s in range(_SPLITS):
        out = _plane_transpose(fs[s], out, s * ls, l, b)
    return jnp.transpose(out, (2, 0, 1))


# speedup vs baseline: 1.0598x; 1.0150x over previous
import functools

import jax
import jax.numpy as jnp
import numpy as np
from jax import lax
from jax.experimental import pallas as pl
from jax.experimental.pallas import tpu as pltpu
from jax.experimental.pallas import tpu_sc as plsc

DIM = 64
MAX_NORM = 1.0


_Q = 2048

_HI = np.uint32(0xFFFF0000)


def _sel(parity):
    d = lax.broadcasted_iota(jnp.int32, (DIM, DIM // 2), 0)
    c = lax.broadcasted_iota(jnp.int32, (DIM, DIM // 2), 1)
    return (d == 2 * c + parity).astype(jnp.float32)


def _pack_bf16_pairs(even_f32, odd_f32):
    eb = even_f32.astype(jnp.bfloat16).astype(jnp.float32)
    ob = odd_f32.astype(jnp.bfloat16).astype(jnp.float32)
    ue = lax.bitcast_convert_type(eb, jnp.uint32)
    uo = lax.bitcast_convert_type(ob, jnp.uint32)
    packed = jnp.bitwise_or(jnp.right_shift(ue, 16),
                            jnp.bitwise_and(uo, _HI))
    return lax.bitcast_convert_type(packed, jnp.float32)


def _renorm_body(tt_ref, out_ref):
    x = tt_ref[...]
    norm = jnp.sqrt(jnp.sum(x * x, axis=0, keepdims=True))
    scale = jnp.where(norm > MAX_NORM, MAX_NORM / (norm + 1e-7), 1.0)
    y = x * scale
    se, so = _sel(0), _sel(1)
    dn = (((0,), (0,)), ((), ()))
    quarters = []
    for j in range(4):
        q = y[:, j * _Q:(j + 1) * _Q]
        ye = lax.dot_general(q, se, dn, preferred_element_type=jnp.float32)
        yo = lax.dot_general(q, so, dn, preferred_element_type=jnp.float32)
        quarters.append(_pack_bf16_pairs(ye, yo))
    out_ref[...] = jnp.concatenate(quarters, axis=1)


def _renorm_flat(table):
    vocab = table.shape[0]
    g = (vocab + 4 * _Q - 1) // (4 * _Q)
    sf = pl.pallas_call(
        _renorm_body,
        grid=(g,),
        in_specs=[pl.BlockSpec((DIM, 4 * _Q), lambda i: (0, i))],
        out_specs=pl.BlockSpec((_Q, 2 * DIM), lambda i: (i, 0)),
        out_shape=jax.ShapeDtypeStruct((g * _Q, 2 * DIM), jnp.float32),
    )(table.T)
    return sf.reshape(g * _Q * 4, DIM // 2)


_CHUNK = 128
_NBUF = 8
_LAG = _NBUF // 2


@functools.cache
def _make_gather(bsz, seq, row_off):
    info = plsc.get_sparse_core_info()
    nc, ns = info.num_cores, info.num_subcores
    nw = nc * ns
    bblks = bsz // _CHUNK
    qrows = bsz // 4
    per_w = bblks * seq // nw
    assert per_w * nw == bblks * seq and bblks * _CHUNK == bsz
    assert bblks & (bblks - 1) == 0
    sh_l = bblks.bit_length() - 1
    sh_b = _CHUNK.bit_length() - 1
    sh_q = qrows.bit_length() - 1
    mesh = plsc.VectorSubcoreMesh(core_axis_name="c", subcore_axis_name="s")

    @functools.partial(
        pl.kernel,
        mesh=mesh,
        compiler_params=pltpu.CompilerParams(
            use_tc_tiling_on_sc=False, needs_layout_passes=False),
        out_type=jax.ShapeDtypeStruct((seq * qrows, 2 * DIM), jnp.float32),
        scratch_types=(
            [pltpu.VMEM((per_w, _CHUNK), jnp.int32)]
            + [pltpu.VMEM((_CHUNK, DIM // 2), jnp.float32)
               for _ in range(_NBUF)]
            + [pltpu.SemaphoreType.DMA for _ in range(2 * _NBUF)]
        ),
    )
    def gather_k(tab_hbm, idx_hbm, out_hbm, idx_v, *rest):
        bufs = rest[:_NBUF]
        gsems = rest[_NBUF:2 * _NBUF]
        wsems = rest[2 * _NBUF:]
        wid = lax.axis_index("s") * nc + lax.axis_index("c")
        ubase = wid * per_w
        pltpu.sync_copy(
            idx_hbm.at[pl.ds(row_off + wid * per_w, per_w)], idx_v)

        qb = _Q.bit_length() - 1
        @pl.loop(0, per_w)
        def _(j):
            for k in range(_CHUNK // 16):
                v = idx_v[j, pl.ds(k * 16, 16)]
                g = jnp.right_shift(v, qb + 2)
                jj = jnp.bitwise_and(jnp.right_shift(v, qb), 3)
                p = jnp.bitwise_and(v, _Q - 1)
                idx_v[j, pl.ds(k * 16, 16)] = (
                    jnp.left_shift(g, qb + 2) + jnp.left_shift(p, 2) + jj)

        hg = [None] * _NBUF
        hw = [None] * _NBUF
        for j in range(per_w + _LAG):
            if j < per_w:
                b = j % _NBUF
                if j >= _NBUF:
                    hw[b].wait()
                hg[b] = pltpu.async_copy(
                    tab_hbm.at[idx_v.at[j]], bufs[b], gsems[b])
            i = j - _LAG
            if 0 <= i < per_w:
                bi = i % _NBUF
                hg[bi].wait()
                u = ubase + i
                li = jnp.right_shift(u, sh_l)
                b0 = jnp.left_shift(jnp.bitwise_and(u, bblks - 1), sh_b)
                seg = jnp.right_shift(b0, sh_q)
                row0 = pl.multiple_of(
                    li * qrows + jnp.bitwise_and(b0, qrows - 1), _CHUNK)
                col0 = pl.multiple_of(seg * (DIM // 2), DIM // 2)
                hw[bi] = pltpu.async_copy(
                    bufs[bi],
                    out_hbm.at[pl.ds(row0, _CHUNK), pl.ds(col0, DIM // 2)],
                    wsems[bi])
        for i in range(max(0, per_w - _NBUF), per_w):
            hw[i % _NBUF].wait()

    return gather_k


_PL3 = 5


def _interleave_mat():
    d = lax.broadcasted_iota(jnp.int32, (DIM, DIM), 0)
    c = lax.broadcasted_iota(jnp.int32, (DIM, DIM), 1)
    return (((c < 32) & (d == 2 * c))
            | ((c >= 32) & (d == 2 * (c - 32) + 1))).astype(jnp.float32)


def _plane_body(f_ref, *rest):
    out_ref = rest[-1]
    x = f_ref[...]
    qrows = x.shape[0] // _PL3
    u_mat = _interleave_mat()
    dn = (((1,), (1,)), ((), ()))
    planes = []
    for p in range(_PL3):
        xp = x[p * qrows:(p + 1) * qrows]
        segs = []
        for q in range(4):
            w = lax.bitcast_convert_type(
                xp[:, q * 32:(q + 1) * 32], jnp.uint32)
            lo = lax.bitcast_convert_type(
                jnp.left_shift(w, 16), jnp.float32)
            hi = lax.bitcast_convert_type(
                jnp.bitwise_and(w, _HI), jnp.float32)
            cat = jnp.concatenate([lo, hi], axis=1)
            segs.append(lax.dot_general(
                u_mat, cat, dn, preferred_element_type=jnp.float32))
        planes.append(jnp.concatenate(segs, axis=1)[None])
    out_ref[...] = jnp.concatenate(planes, axis=0)


def _plane_transpose(f, prev, l_off, seq, bsz):
    qrows = bsz // 4
    blk_off = l_off // _PL3
    in_specs = [pl.BlockSpec((_PL3 * qrows, 2 * DIM), lambda i: (i, 0))]
    args = (f,)
    aliases = {}
    if prev is not None:
        in_specs.append(pl.BlockSpec(memory_space=pl.ANY))
        args = (f, prev)
        aliases = {1: 0}
    return pl.pallas_call(
        _plane_body,
        grid=(f.shape[0] // (_PL3 * qrows),),
        in_specs=in_specs,
        out_specs=pl.BlockSpec(
            (_PL3, DIM, bsz), lambda i: (i + blk_off, 0, 0)),
        out_shape=jax.ShapeDtypeStruct((seq, DIM, bsz), jnp.float32),
        input_output_aliases=aliases,
    )(*args)


_SPLITS = 2


def kernel(doc, table):
    b, l = doc.shape
    flat = _renorm_flat(table)
    idx2d = doc.T.reshape(b * l // _CHUNK, _CHUNK)
    ls = l // _SPLITS
    rows_per_split = ls * b // _CHUNK
    fs = [_make_gather(b, ls, s * rows_per_split)(flat, idx2d)
          for s in range(_SPLITS)]
    out = None
    for s in range(_SPLITS):
        out = _plane_transpose(fs[s], out, s * ls, l, b)
    return jnp.transpose(out, (2, 0, 1))
